# baseline (speedup 1.0000x reference)
import functools

import jax
import jax.numpy as jnp
from jax import lax
from jax.experimental import pallas as pl
from jax.experimental.pallas import tpu as pltpu
from jax.experimental.pallas import tpu_sc as plsc

N_NODES = 10000
N_EDGES = 320000
D = 128

NC = 2
NS = 16
NW = NC * NS

CH = 128
N_CHUNKS = N_EDGES // CH
CPW = N_CHUNKS // NW
N_EXTRA = N_CHUNKS - CPW * NW

ROWS_PER_SUB = 624
ROWS_TAIL = N_NODES - NS * ROWS_PER_SUB

N_PAD = 16384
DROWS = N_PAD // NS


def _worker_id():
    return lax.axis_index("s") * NC + lax.axis_index("c")


DEG_WIN = 8


def _deg_body(dst3d_hbm, ones_hbm, zeros_hbm, out0_hbm, out1_hbm,
              dst_all, ones_v, deg_sh, ssem):
    cid = lax.axis_index("c")
    sid = lax.axis_index("s")
    w = _worker_id()

    base = sid * DROWS
    pltpu.sync_copy(zeros_hbm, deg_sh.at[pl.ds(base, DROWS)])
    pltpu.sync_copy(ones_hbm, ones_v)
    pltpu.sync_copy(dst3d_hbm.at[pl.ds(w * CPW, CPW)], dst_all)
    plsc.subcore_barrier()

    def _swait():
        pltpu.make_async_copy(ones_v, deg_sh.at[dst_all.at[0, 0]],
                              ssem).wait()

    def chunk_step(j, _):
        pltpu.async_copy(ones_v, deg_sh.at[dst_all.at[j, 0]], ssem, add=True)

        @pl.when(j >= DEG_WIN)
        def _():
            _swait()
        return 0

    lax.fori_loop(0, CPW, chunk_step, 0)

    @pl.when(w < N_EXTRA)
    def _():
        pltpu.sync_copy(dst3d_hbm.at[NW * CPW + w], dst_all.at[0])
        pltpu.async_copy(ones_v, deg_sh.at[dst_all.at[0, 0]], ssem, add=True)
        _swait()

    lax.fori_loop(0, DEG_WIN, lambda j, _: (_swait(), 0)[1], 0)

    plsc.subcore_barrier()

    for c, out_hbm in enumerate((out0_hbm, out1_hbm)):
        @pl.when(cid == c)
        def _(out_hbm=out_hbm):
            pltpu.sync_copy(deg_sh.at[pl.ds(base, DROWS)],
                            out_hbm.at[pl.ds(base, DROWS)])


@functools.cache
def _deg_call():
    return pl.kernel(
        _deg_body,
        out_type=[jax.ShapeDtypeStruct((N_PAD,), jnp.float32),
                  jax.ShapeDtypeStruct((N_PAD,), jnp.float32)],
        mesh=plsc.VectorSubcoreMesh(core_axis_name="c", subcore_axis_name="s",
                                    num_cores=NC, num_subcores=NS),
        scratch_types=[
            pltpu.VMEM((CPW, 1, CH), jnp.int32),
            pltpu.VMEM((CH,), jnp.float32),
            pltpu.VMEM_SHARED((N_PAD,), jnp.float32),
            pltpu.SemaphoreType.DMA,
        ],
    )


NRB = 3
NIS = 6
N_MACRO = 12


def _agg_body(g_hbm, e2_hbm, zeros_hbm, out_hbm,
              idx_ring, rows0, rows1, rows2, agg_sh,
              isem0, isem1, isem2, isem3, isem4, isem5,
              gsem0, gsem1, gsem2, ssem0, ssem1, ssem2):
    cid = lax.axis_index("c")
    sid = lax.axis_index("s")
    w = _worker_id()
    rows = (rows0, rows1, rows2)
    isems = (isem0, isem1, isem2, isem3, isem4, isem5)
    gsems = (gsem0, gsem1, gsem2)
    ssems = (ssem0, ssem1, ssem2)

    base = sid * ROWS_PER_SUB
    pltpu.sync_copy(zeros_hbm, agg_sh.at[pl.ds(base, ROWS_PER_SUB)])

    @pl.when(sid == NS - 1)
    def _():
        pltpu.sync_copy(zeros_hbm.at[pl.ds(0, ROWS_TAIL)],
                        agg_sh.at[pl.ds(N_NODES - ROWS_TAIL, ROWS_TAIL)])

    plsc.subcore_barrier()

    cbase = w * CPW

    def _idx_issue(c, s):
        pltpu.async_copy(e2_hbm.at[cbase + c], idx_ring.at[s], isems[s])

    def _idx_wait(s):
        pltpu.make_async_copy(e2_hbm.at[0], idx_ring.at[s], isems[s]).wait()

    H = CH // 2

    def _gather(s, r):
        pltpu.async_copy(g_hbm.at[idx_ring.at[s, 0, pl.ds(0, H)]],
                         rows[r].at[pl.ds(0, H)], gsems[r])
        pltpu.async_copy(g_hbm.at[idx_ring.at[s, 0, pl.ds(H, H)]],
                         rows[r].at[pl.ds(H, H)], gsems[r])

    def _gwait(r):
        pltpu.make_async_copy(g_hbm.at[idx_ring.at[0, 0, pl.ds(0, H)]],
                              rows[r].at[pl.ds(0, H)], gsems[r]).wait()
        pltpu.make_async_copy(g_hbm.at[idx_ring.at[0, 0, pl.ds(H, H)]],
                              rows[r].at[pl.ds(H, H)], gsems[r]).wait()

    def _scat(s, r):
        pltpu.async_copy(rows[r], agg_sh.at[idx_ring.at[s, 1]], ssems[r],
                         add=True)

    def _swait(r):
        pltpu.make_async_copy(rows[r], agg_sh.at[idx_ring.at[0, 1]],
                              ssems[r]).wait()

    def _triplet(c_mod6, jj=None, c_static=None, first=False):
        t = c_mod6 % 6
        sA, sB, sC = (t + 2) % 6, (t + 1) % 6, t
        rB, rC = (t + 1) % 3, t % 3
        if c_static is None:
            cA = 6 * jj + t + 2
            cB = 6 * jj + t + 1
            has_A = has_B = True
        else:
            cA, cB = c_static + 2, c_static + 1
            has_A, has_B = cA <= CPW - 1, cB <= CPW - 1
        if has_A:
            _idx_issue(cA, sA)
        if has_B:
            _idx_wait(sB)
            if first:
                @pl.when(jj > 0)
                def _():
                    _swait(rB)
            elif c_static is None or cB >= 3:
                _swait(rB)
            _gather(sB, rB)
        _gwait(rC)
        _scat(sC, rC)

    _idx_issue(0, 0)
    _idx_issue(1, 1)
    _idx_wait(0)
    _gather(0, 0)

    def macro(jj, _):
        for t in range(6):
            _triplet(t, jj=jj, first=(t < 2))
        return 0

    lax.fori_loop(0, N_MACRO, macro, 0)
    for c in range(6 * N_MACRO, CPW):
        _triplet(c % 6, c_static=c)
    _swait((CPW - 3) % 3)
    _swait((CPW - 2) % 3)
    _swait((CPW - 1) % 3)

    @pl.when(w < N_EXTRA)
    def _():
        pltpu.sync_copy(e2_hbm.at[NW * CPW + w], idx_ring.at[0])
        pltpu.async_copy(g_hbm.at[idx_ring.at[0, 0]], rows0, gsem0).wait()
        pltpu.sync_copy(rows0, agg_sh.at[idx_ring.at[0, 1]], add=True)

    plsc.subcore_barrier()

    pltpu.sync_copy(agg_sh.at[pl.ds(base, ROWS_PER_SUB)],
                    out_hbm.at[cid, pl.ds(base, ROWS_PER_SUB)])

    @pl.when(sid == NS - 1)
    def _():
        pltpu.sync_copy(agg_sh.at[pl.ds(N_NODES - ROWS_TAIL, ROWS_TAIL)],
                        out_hbm.at[cid, pl.ds(N_NODES - ROWS_TAIL, ROWS_TAIL)])


@functools.cache
def _agg_call():
    return pl.kernel(
        _agg_body,
        out_type=jax.ShapeDtypeStruct((NC, N_NODES, D), jnp.float32),
        mesh=plsc.VectorSubcoreMesh(core_axis_name="c", subcore_axis_name="s",
                                    num_cores=NC, num_subcores=NS),
        scratch_types=[
            pltpu.VMEM((NIS, 2, CH), jnp.int32),
            pltpu.VMEM((CH, D), jnp.float32),
            pltpu.VMEM((CH, D), jnp.float32),
            pltpu.VMEM((CH, D), jnp.float32),
            pltpu.VMEM_SHARED((N_NODES, D), jnp.float32),
        ] + [pltpu.SemaphoreType.DMA] * 12,
    )


RB = 1000
GRID = N_NODES // RB


def _dinv(degp_ref):
    deg = degp_ref[:, 0] + degp_ref[:, 1] + 1.0
    return lax.rsqrt(deg)[:, None]


def _mm1_body(x_ref, w_ref, degp_ref, o_ref):
    o_ref[...] = jnp.dot(x_ref[...] * _dinv(degp_ref), w_ref[...],
                         preferred_element_type=jnp.float32)


_mm1_call = pl.pallas_call(
    _mm1_body,
    grid=(GRID,),
    in_specs=[
        pl.BlockSpec((RB, D), lambda i: (i, 0)),
        pl.BlockSpec((D, D), lambda i: (0, 0)),
        pl.BlockSpec((RB, NC), lambda i: (i, 0)),
    ],
    out_specs=pl.BlockSpec((RB, D), lambda i: (i, 0)),
    out_shape=jax.ShapeDtypeStruct((N_NODES, D), jnp.float32),
)


def _mm2_body(sp_ref, g_ref, degp_ref, b_ref, w_ref, o_ref):
    dinv = _dinv(degp_ref)
    x2 = dinv * (sp_ref[0] + sp_ref[1] + g_ref[...]) + b_ref[...]
    x2 = jnp.maximum(x2, 0.0)
    o_ref[...] = jnp.dot(x2 * dinv, w_ref[...],
                         preferred_element_type=jnp.float32)


_mm2_call = pl.pallas_call(
    _mm2_body,
    grid=(GRID,),
    in_specs=[
        pl.BlockSpec((NC, RB, D), lambda i: (0, i, 0)),
        pl.BlockSpec((RB, D), lambda i: (i, 0)),
        pl.BlockSpec((RB, NC), lambda i: (i, 0)),
        pl.BlockSpec((1, D), lambda i: (0, 0)),
        pl.BlockSpec((D, D), lambda i: (0, 0)),
    ],
    out_specs=pl.BlockSpec((RB, D), lambda i: (i, 0)),
    out_shape=jax.ShapeDtypeStruct((N_NODES, D), jnp.float32),
)


def _fin_body(sp_ref, g_ref, degp_ref, b_ref, o_ref):
    dinv = _dinv(degp_ref)
    o_ref[...] = dinv * (sp_ref[0] + sp_ref[1] + g_ref[...]) + b_ref[...]


_fin_call = pl.pallas_call(
    _fin_body,
    grid=(GRID,),
    in_specs=[
        pl.BlockSpec((NC, RB, D), lambda i: (0, i, 0)),
        pl.BlockSpec((RB, D), lambda i: (i, 0)),
        pl.BlockSpec((RB, NC), lambda i: (i, 0)),
        pl.BlockSpec((1, D), lambda i: (0, 0)),
    ],
    out_specs=pl.BlockSpec((RB, D), lambda i: (i, 0)),
    out_shape=jax.ShapeDtypeStruct((N_NODES, D), jnp.float32),
)


def kernel(x_piece, edge_index_piece, W1, b1, W2, b2):
    ei = edge_index_piece.astype(jnp.int32)
    dst3d = ei[1].reshape(N_CHUNKS, 1, CH)
    e2 = ei.reshape(2, N_CHUNKS, CH).transpose(1, 0, 2)

    ones_e = jnp.ones((CH,), jnp.float32)
    zeros_v = jnp.zeros((DROWS,), jnp.float32)
    zeros_m = jnp.zeros((ROWS_PER_SUB, D), jnp.float32)

    deg0, deg1 = _deg_call()(dst3d, ones_e, zeros_v)
    deg_t = jnp.stack([deg0[:N_NODES], deg1[:N_NODES]], axis=1)

    g1 = _mm1_call(x_piece, W1, deg_t)
    s1 = _agg_call()(g1, e2, zeros_m)
    g2 = _mm2_call(s1, g1, deg_t, b1.reshape(1, D), W2)
    s2 = _agg_call()(g2, e2, zeros_m)
    out = _fin_call(s2, g2, deg_t, b2.reshape(1, D))
    return out


# speedup vs baseline: 1.0084x; 1.0084x over previous
import functools

import jax
import jax.numpy as jnp
from jax import lax
from jax.experimental import pallas as pl
from jax.experimental.pallas import tpu as pltpu
from jax.experimental.pallas import tpu_sc as plsc

N_NODES = 10000
N_EDGES = 320000
D = 128

NC = 2
NS = 16
NW = NC * NS

CH = 128
N_CHUNKS = N_EDGES // CH
CPW = N_CHUNKS // NW
N_EXTRA = N_CHUNKS - CPW * NW

ROWS_PER_SUB = 624
ROWS_TAIL = N_NODES - NS * ROWS_PER_SUB

N_PAD = 16384
DROWS = N_PAD // NS


def _worker_id():
    return lax.axis_index("s") * NC + lax.axis_index("c")


DEG_WIN = 8


def _deg_body(dst3d_hbm, ones_hbm, zeros_hbm, out0_hbm, out1_hbm,
              dst_all, ones_v, deg_sh, ssem):
    cid = lax.axis_index("c")
    sid = lax.axis_index("s")
    w = _worker_id()

    base = sid * DROWS
    pltpu.sync_copy(zeros_hbm, deg_sh.at[pl.ds(base, DROWS)])
    pltpu.sync_copy(ones_hbm, ones_v)
    pltpu.sync_copy(dst3d_hbm.at[pl.ds(w * CPW, CPW)], dst_all)
    plsc.subcore_barrier()

    def _swait():
        pltpu.make_async_copy(ones_v, deg_sh.at[dst_all.at[0, 0]],
                              ssem).wait()

    def chunk_step(j, _):
        pltpu.async_copy(ones_v, deg_sh.at[dst_all.at[j, 0]], ssem, add=True)

        @pl.when(j >= DEG_WIN)
        def _():
            _swait()
        return 0

    lax.fori_loop(0, CPW, chunk_step, 0)

    @pl.when(w < N_EXTRA)
    def _():
        pltpu.sync_copy(dst3d_hbm.at[NW * CPW + w], dst_all.at[0])
        pltpu.async_copy(ones_v, deg_sh.at[dst_all.at[0, 0]], ssem, add=True)
        _swait()

    lax.fori_loop(0, DEG_WIN, lambda j, _: (_swait(), 0)[1], 0)

    plsc.subcore_barrier()

    for c, out_hbm in enumerate((out0_hbm, out1_hbm)):
        @pl.when(cid == c)
        def _(out_hbm=out_hbm):
            pltpu.sync_copy(deg_sh.at[pl.ds(base, DROWS)],
                            out_hbm.at[pl.ds(base, DROWS)])


@functools.cache
def _deg_call():
    return pl.kernel(
        _deg_body,
        out_type=[jax.ShapeDtypeStruct((N_PAD,), jnp.float32),
                  jax.ShapeDtypeStruct((N_PAD,), jnp.float32)],
        mesh=plsc.VectorSubcoreMesh(core_axis_name="c", subcore_axis_name="s",
                                    num_cores=NC, num_subcores=NS),
        scratch_types=[
            pltpu.VMEM((CPW, 1, CH), jnp.int32),
            pltpu.VMEM((CH,), jnp.float32),
            pltpu.VMEM_SHARED((N_PAD,), jnp.float32),
            pltpu.SemaphoreType.DMA,
        ],
    )


NRB = 3
NIS = 6
N_MACRO = 12


def _agg_body(g_hbm, e2_hbm, zeros_hbm, out_hbm,
              idx_ring, rows0, rows1, rows2, agg_sh,
              isem0, isem1, isem2, isem3, isem4, isem5,
              gsem0, gsem1, gsem2, ssem0, ssem1, ssem2):
    cid = lax.axis_index("c")
    sid = lax.axis_index("s")
    w = _worker_id()
    rows = (rows0, rows1, rows2)
    isems = (isem0, isem1, isem2, isem3, isem4, isem5)
    gsems = (gsem0, gsem1, gsem2)
    ssems = (ssem0, ssem1, ssem2)

    base = sid * ROWS_PER_SUB
    pltpu.sync_copy(zeros_hbm, agg_sh.at[pl.ds(base, ROWS_PER_SUB)])

    @pl.when(sid == NS - 1)
    def _():
        pltpu.sync_copy(zeros_hbm.at[pl.ds(0, ROWS_TAIL)],
                        agg_sh.at[pl.ds(N_NODES - ROWS_TAIL, ROWS_TAIL)])

    plsc.subcore_barrier()

    cbase = w * CPW

    def _idx_issue(c, s):
        pltpu.async_copy(e2_hbm.at[cbase + c], idx_ring.at[s], isems[s])

    def _idx_wait(s):
        pltpu.make_async_copy(e2_hbm.at[0], idx_ring.at[s], isems[s]).wait()

    def _gather(s, r):
        pltpu.async_copy(g_hbm.at[idx_ring.at[s, 0]], rows[r], gsems[r])

    def _gwait(r):
        pltpu.make_async_copy(g_hbm.at[idx_ring.at[0, 0]], rows[r],
                              gsems[r]).wait()

    def _scat(s, r):
        pltpu.async_copy(rows[r], agg_sh.at[idx_ring.at[s, 1]], ssems[r],
                         add=True)

    def _swait(r):
        pltpu.make_async_copy(rows[r], agg_sh.at[idx_ring.at[0, 1]],
                              ssems[r]).wait()

    def _triplet(c_mod6, jj=None, c_static=None, first=False):
        t = c_mod6 % 6
        sA, sB, sC = (t + 2) % 6, (t + 1) % 6, t
        rB, rC = (t + 1) % 3, t % 3
        if c_static is None:
            cA = 6 * jj + t + 2
            cB = 6 * jj + t + 1
            has_A = has_B = True
        else:
            cA, cB = c_static + 2, c_static + 1
            has_A, has_B = cA <= CPW - 1, cB <= CPW - 1
        if has_A:
            _idx_issue(cA, sA)
        if has_B:
            _idx_wait(sB)
            if first:
                @pl.when(jj > 0)
                def _():
                    _swait(rB)
            elif c_static is None or cB >= 3:
                _swait(rB)
            _gather(sB, rB)
        _gwait(rC)
        _scat(sC, rC)

    _idx_issue(0, 0)
    _idx_issue(1, 1)
    _idx_wait(0)
    _gather(0, 0)

    def macro(jj, _):
        for t in range(6):
            _triplet(t, jj=jj, first=(t < 2))
        return 0

    lax.fori_loop(0, N_MACRO, macro, 0)
    for c in range(6 * N_MACRO, CPW):
        _triplet(c % 6, c_static=c)
    _swait((CPW - 3) % 3)
    _swait((CPW - 2) % 3)
    _swait((CPW - 1) % 3)

    @pl.when(w < N_EXTRA)
    def _():
        pltpu.sync_copy(e2_hbm.at[NW * CPW + w], idx_ring.at[0])
        pltpu.async_copy(g_hbm.at[idx_ring.at[0, 0]], rows0, gsem0).wait()
        pltpu.sync_copy(rows0, agg_sh.at[idx_ring.at[0, 1]], add=True)

    plsc.subcore_barrier()

    pltpu.sync_copy(agg_sh.at[pl.ds(base, ROWS_PER_SUB)],
                    out_hbm.at[cid, pl.ds(base, ROWS_PER_SUB)])

    @pl.when(sid == NS - 1)
    def _():
        pltpu.sync_copy(agg_sh.at[pl.ds(N_NODES - ROWS_TAIL, ROWS_TAIL)],
                        out_hbm.at[cid, pl.ds(N_NODES - ROWS_TAIL, ROWS_TAIL)])


@functools.cache
def _agg_call():
    return pl.kernel(
        _agg_body,
        out_type=jax.ShapeDtypeStruct((NC, N_NODES, D), jnp.float32),
        mesh=plsc.VectorSubcoreMesh(core_axis_name="c", subcore_axis_name="s",
                                    num_cores=NC, num_subcores=NS),
        scratch_types=[
            pltpu.VMEM((NIS, 2, CH), jnp.int32),
            pltpu.VMEM((CH, D), jnp.float32),
            pltpu.VMEM((CH, D), jnp.float32),
            pltpu.VMEM((CH, D), jnp.float32),
            pltpu.VMEM_SHARED((N_NODES, D), jnp.float32),
        ] + [pltpu.SemaphoreType.DMA] * 12,
    )


RB = 1000
GRID = N_NODES // RB


def _dinv(degp_ref):
    deg = degp_ref[:, 0] + degp_ref[:, 1] + 1.0
    return lax.rsqrt(deg)[:, None]


def _mm1_body(x_ref, w_ref, degp_ref, o_ref):
    o_ref[...] = jnp.dot(x_ref[...] * _dinv(degp_ref), w_ref[...],
                         preferred_element_type=jnp.float32)


_mm1_call = pl.pallas_call(
    _mm1_body,
    grid=(GRID,),
    in_specs=[
        pl.BlockSpec((RB, D), lambda i: (i, 0)),
        pl.BlockSpec((D, D), lambda i: (0, 0)),
        pl.BlockSpec((RB, NC), lambda i: (i, 0)),
    ],
    out_specs=pl.BlockSpec((RB, D), lambda i: (i, 0)),
    out_shape=jax.ShapeDtypeStruct((N_NODES, D), jnp.float32),
)


def _mm2_body(sp_ref, g_ref, degp_ref, b_ref, w_ref, o_ref):
    dinv = _dinv(degp_ref)
    x2 = dinv * (sp_ref[0] + sp_ref[1] + g_ref[...]) + b_ref[...]
    x2 = jnp.maximum(x2, 0.0)
    o_ref[...] = jnp.dot(x2 * dinv, w_ref[...],
                         preferred_element_type=jnp.float32)


_mm2_call = pl.pallas_call(
    _mm2_body,
    grid=(GRID,),
    in_specs=[
        pl.BlockSpec((NC, RB, D), lambda i: (0, i, 0)),
        pl.BlockSpec((RB, D), lambda i: (i, 0)),
        pl.BlockSpec((RB, NC), lambda i: (i, 0)),
        pl.BlockSpec((1, D), lambda i: (0, 0)),
        pl.BlockSpec((D, D), lambda i: (0, 0)),
    ],
    out_specs=pl.BlockSpec((RB, D), lambda i: (i, 0)),
    out_shape=jax.ShapeDtypeStruct((N_NODES, D), jnp.float32),
)


def _fin_body(sp_ref, g_ref, degp_ref, b_ref, o_ref):
    dinv = _dinv(degp_ref)
    o_ref[...] = dinv * (sp_ref[0] + sp_ref[1] + g_ref[...]) + b_ref[...]


_fin_call = pl.pallas_call(
    _fin_body,
    grid=(GRID,),
    in_specs=[
        pl.BlockSpec((NC, RB, D), lambda i: (0, i, 0)),
        pl.BlockSpec((RB, D), lambda i: (i, 0)),
        pl.BlockSpec((RB, NC), lambda i: (i, 0)),
        pl.BlockSpec((1, D), lambda i: (0, 0)),
    ],
    out_specs=pl.BlockSpec((RB, D), lambda i: (i, 0)),
    out_shape=jax.ShapeDtypeStruct((N_NODES, D), jnp.float32),
)


def kernel(x_piece, edge_index_piece, W1, b1, W2, b2):
    ei = edge_index_piece.astype(jnp.int32)
    dst3d = ei[1].reshape(N_CHUNKS, 1, CH)
    e2 = ei.reshape(2, N_CHUNKS, CH).transpose(1, 0, 2)

    ones_e = jnp.ones((CH,), jnp.float32)
    zeros_v = jnp.zeros((DROWS,), jnp.float32)
    zeros_m = jnp.zeros((ROWS_PER_SUB, D), jnp.float32)

    deg0, deg1 = _deg_call()(dst3d, ones_e, zeros_v)
    deg_t = jnp.stack([deg0[:N_NODES], deg1[:N_NODES]], axis=1)

    g1 = _mm1_call(x_piece, W1, deg_t)
    s1 = _agg_call()(g1, e2, zeros_m)
    g2 = _mm2_call(s1, g1, deg_t, b1.reshape(1, D), W2)
    s2 = _agg_call()(g2, e2, zeros_m)
    out = _fin_call(s2, g2, deg_t, b2.reshape(1, D))
    return out
